# Initial kernel scaffold; baseline (speedup 1.0000x reference)
import jax
import jax.numpy as jnp
from jax.experimental import pallas as pl


def kernel(x, z, edge_src, edge_dst, edge_attr, edge_scalars, W_lin1, W_fc1, W_fc2, W_lin2_s, W_lin2_v, W_sc):
    raise NotImplementedError("write your pallas kernel here")


# speedup vs baseline: 1.5598x; 1.5598x over previous
import functools
import math

import jax
import jax.numpy as jnp
from jax.experimental import pallas as pl
from jax.experimental.pallas import tpu as pltpu

_INV_SQRT128 = 1.0 / math.sqrt(128.0)
_INV_SQRT12 = 1.0 / math.sqrt(12.0)
_INV_SQRT100 = 1.0 / math.sqrt(100.0)
_INV_SQRT_NN = 1.0 / math.sqrt(16.0)

N_BLK = 2000
E_BLK = 2000


def _node_body(x_ref, z_ref, wl_ref, ws_ref, xl_ref, sc_ref):
    xz = x_ref[...] * z_ref[...]
    xl_ref[...] = (xz @ wl_ref[...]) * _INV_SQRT128
    sc_ref[...] = (xz @ ws_ref[...]) * _INV_SQRT128


def _node_stage(x, z, W_lin1, W_sc):
    n = x.shape[0]
    grid = (n // N_BLK,)
    return pl.pallas_call(
        _node_body,
        grid=grid,
        in_specs=[
            pl.BlockSpec((N_BLK, 128), lambda i: (i, 0)),
            pl.BlockSpec((N_BLK, 1), lambda i: (i, 0)),
            pl.BlockSpec((128, 128), lambda i: (0, 0)),
            pl.BlockSpec((128, 96), lambda i: (0, 0)),
        ],
        out_specs=[
            pl.BlockSpec((N_BLK, 128), lambda i: (i, 0)),
            pl.BlockSpec((N_BLK, 96), lambda i: (i, 0)),
        ],
        out_shape=[
            jax.ShapeDtypeStruct((n, 128), jnp.float32),
            jax.ShapeDtypeStruct((n, 96), jnp.float32),
        ],
    )(x, z, W_lin1, W_sc)


def _edge_body(es_ref, xs_ref, attr_ref, wf1_ref, wf2_ref, wls_ref, wvr_ref,
               r3_ref, msgs_ref, msgv_ref):
    h = jax.nn.silu((es_ref[...] @ wf1_ref[...]) * _INV_SQRT12)
    w = (h @ wf2_ref[...]) * _INV_SQRT100
    xs = xs_ref[...]
    sh0 = attr_ref[:, 0:1]
    sh1 = attr_ref[:, 1:4]
    t0 = xs * sh0 * w[:, :128]
    msgs_ref[...] = (t0 @ wls_ref[...]) * _INV_SQRT128
    t1 = xs * w[:, 128:]
    msgv_ref[...] = ((t1 @ wvr_ref[...]) * _INV_SQRT128) * (sh1 @ r3_ref[...])


def _edge_stage(es, xs, attr, W_fc1, W_fc2, W_lin2_s, Wv_rep, R3):
    e = es.shape[0]
    grid = (e // E_BLK,)
    return pl.pallas_call(
        _edge_body,
        grid=grid,
        in_specs=[
            pl.BlockSpec((E_BLK, 12), lambda i: (i, 0)),
            pl.BlockSpec((E_BLK, 128), lambda i: (i, 0)),
            pl.BlockSpec((E_BLK, 4), lambda i: (i, 0)),
            pl.BlockSpec((12, 100), lambda i: (0, 0)),
            pl.BlockSpec((100, 256), lambda i: (0, 0)),
            pl.BlockSpec((128, 96), lambda i: (0, 0)),
            pl.BlockSpec((128, 96), lambda i: (0, 0)),
            pl.BlockSpec((3, 96), lambda i: (0, 0)),
        ],
        out_specs=[
            pl.BlockSpec((E_BLK, 96), lambda i: (i, 0)),
            pl.BlockSpec((E_BLK, 96), lambda i: (i, 0)),
        ],
        out_shape=[
            jax.ShapeDtypeStruct((e, 96), jnp.float32),
            jax.ShapeDtypeStruct((e, 96), jnp.float32),
        ],
    )(es, xs, attr, W_fc1, W_fc2, W_lin2_s, Wv_rep, R3)


def _final_body(sc_ref, aggs_ref, aggv_ref, z_ref, e32_ref, out_ref):
    z = z_ref[...]
    s_out = sc_ref[...] + aggs_ref[...] * z * _INV_SQRT_NN
    scal = jax.nn.silu(s_out[:, :64])
    gates = jax.nn.sigmoid(s_out[:, 64:96])
    gated = (aggv_ref[...] * z * _INV_SQRT_NN) * (gates @ e32_ref[...])
    out_ref[:, :64] = scal
    out_ref[:, 64:160] = gated


def _final_stage(sc, agg_s, agg_v, z, E32):
    n = sc.shape[0]
    grid = (n // N_BLK,)
    return pl.pallas_call(
        _final_body,
        grid=grid,
        in_specs=[
            pl.BlockSpec((N_BLK, 96), lambda i: (i, 0)),
            pl.BlockSpec((N_BLK, 96), lambda i: (i, 0)),
            pl.BlockSpec((N_BLK, 96), lambda i: (i, 0)),
            pl.BlockSpec((N_BLK, 1), lambda i: (i, 0)),
            pl.BlockSpec((32, 96), lambda i: (0, 0)),
        ],
        out_specs=pl.BlockSpec((N_BLK, 160), lambda i: (i, 0)),
        out_shape=jax.ShapeDtypeStruct((n, 160), jnp.float32),
    )(sc, agg_s, agg_v, z, E32)


def kernel(x, z, edge_src, edge_dst, edge_attr, edge_scalars,
           W_lin1, W_fc1, W_fc2, W_lin2_s, W_lin2_v, W_sc):
    n = x.shape[0]
    Wv_rep = jnp.repeat(W_lin2_v, 3, axis=1)
    R3 = jnp.tile(jnp.eye(3, dtype=jnp.float32), (1, 32))
    E32 = jnp.repeat(jnp.eye(32, dtype=jnp.float32), 3, axis=1)

    xl, sc = _node_stage(x, z, W_lin1, W_sc)
    xs = xl[edge_src]
    msg_s, msg_v = _edge_stage(edge_scalars, xs, edge_attr,
                               W_fc1, W_fc2, W_lin2_s, Wv_rep, R3)
    agg_s = jax.ops.segment_sum(msg_s, edge_dst, num_segments=n)
    agg_v = jax.ops.segment_sum(msg_v, edge_dst, num_segments=n)
    return _final_stage(sc, agg_s, agg_v, z, E32)


# speedup vs baseline: 1.8400x; 1.1796x over previous
import functools
import math

import jax
import jax.numpy as jnp
from jax import lax
from jax.experimental import pallas as pl
from jax.experimental.pallas import tpu as pltpu
from jax.experimental.pallas import tpu_sc as plsc

_INV_SQRT128 = 1.0 / math.sqrt(128.0)
_INV_SQRT12 = 1.0 / math.sqrt(12.0)
_INV_SQRT100 = 1.0 / math.sqrt(100.0)
_INV_SQRT_NN = 1.0 / math.sqrt(16.0)

N_BLK = 2000
E_BLK = 2000


def _node_body(x_ref, z_ref, wl_ref, ws_ref, xl_ref, sc_ref):
    xz = x_ref[...] * z_ref[...]
    xl_ref[...] = (xz @ wl_ref[...]) * _INV_SQRT128
    sc_ref[...] = (xz @ ws_ref[...]) * _INV_SQRT128


def _node_stage(x, z, W_lin1, W_sc):
    n = x.shape[0]
    grid = (n // N_BLK,)
    return pl.pallas_call(
        _node_body,
        grid=grid,
        in_specs=[
            pl.BlockSpec((N_BLK, 128), lambda i: (i, 0)),
            pl.BlockSpec((N_BLK, 1), lambda i: (i, 0)),
            pl.BlockSpec((128, 128), lambda i: (0, 0)),
            pl.BlockSpec((128, 96), lambda i: (0, 0)),
        ],
        out_specs=[
            pl.BlockSpec((N_BLK, 128), lambda i: (i, 0)),
            pl.BlockSpec((N_BLK, 96), lambda i: (i, 0)),
        ],
        out_shape=[
            jax.ShapeDtypeStruct((n, 128), jnp.float32),
            jax.ShapeDtypeStruct((n, 96), jnp.float32),
        ],
    )(x, z, W_lin1, W_sc)


def _edge_body(es_ref, xs_ref, attr_ref, wf1_ref, wf2_ref, wls_ref, wvr_ref,
               r3_ref, msgs_ref, msgv_ref):
    h = jax.nn.silu((es_ref[...] @ wf1_ref[...]) * _INV_SQRT12)
    w = (h @ wf2_ref[...]) * _INV_SQRT100
    xs = xs_ref[...]
    sh0 = attr_ref[:, 0:1]
    sh1 = attr_ref[:, 1:4]
    t0 = xs * sh0 * w[:, :128]
    msgs_ref[...] = (t0 @ wls_ref[...]) * _INV_SQRT128
    t1 = xs * w[:, 128:]
    msgv_ref[...] = ((t1 @ wvr_ref[...]) * _INV_SQRT128) * (sh1 @ r3_ref[...])


def _edge_stage(es, xs, attr, W_fc1, W_fc2, W_lin2_s, Wv_rep, R3):
    e = es.shape[0]
    grid = (e // E_BLK,)
    return pl.pallas_call(
        _edge_body,
        grid=grid,
        in_specs=[
            pl.BlockSpec((E_BLK, 12), lambda i: (i, 0)),
            pl.BlockSpec((E_BLK, 128), lambda i: (i, 0)),
            pl.BlockSpec((E_BLK, 4), lambda i: (i, 0)),
            pl.BlockSpec((12, 100), lambda i: (0, 0)),
            pl.BlockSpec((100, 256), lambda i: (0, 0)),
            pl.BlockSpec((128, 96), lambda i: (0, 0)),
            pl.BlockSpec((128, 96), lambda i: (0, 0)),
            pl.BlockSpec((3, 96), lambda i: (0, 0)),
        ],
        out_specs=[
            pl.BlockSpec((E_BLK, 96), lambda i: (i, 0)),
            pl.BlockSpec((E_BLK, 96), lambda i: (i, 0)),
        ],
        out_shape=[
            jax.ShapeDtypeStruct((e, 96), jnp.float32),
            jax.ShapeDtypeStruct((e, 96), jnp.float32),
        ],
    )(es, xs, attr, W_fc1, W_fc2, W_lin2_s, Wv_rep, R3)


def _final_body(sc_ref, aggs_ref, aggv_ref, z_ref, e32_ref, out_ref):
    z = z_ref[...]
    s_out = sc_ref[...] + aggs_ref[...] * z * _INV_SQRT_NN
    scal = jax.nn.silu(s_out[:, :64])
    gates = jax.nn.sigmoid(s_out[:, 64:96])
    gated = (aggv_ref[...] * z * _INV_SQRT_NN) * (gates @ e32_ref[...])
    out_ref[:, :64] = scal
    out_ref[:, 64:160] = gated


def _final_stage(sc, agg_s, agg_v, z, E32):
    n = sc.shape[0]
    grid = (n // N_BLK,)
    return pl.pallas_call(
        _final_body,
        grid=grid,
        in_specs=[
            pl.BlockSpec((N_BLK, 96), lambda i: (i, 0)),
            pl.BlockSpec((N_BLK, 96), lambda i: (i, 0)),
            pl.BlockSpec((N_BLK, 96), lambda i: (i, 0)),
            pl.BlockSpec((N_BLK, 1), lambda i: (i, 0)),
            pl.BlockSpec((32, 96), lambda i: (0, 0)),
        ],
        out_specs=pl.BlockSpec((N_BLK, 160), lambda i: (i, 0)),
        out_shape=jax.ShapeDtypeStruct((n, 160), jnp.float32),
    )(sc, agg_s, agg_v, z, E32)


_NW = 32
_GCH = 128
_GNBUF = 4


def _sc_gather(xl, src):
    e = src.shape[0]
    d = xl.shape[1]
    n_chunks = e // _GCH
    assert e % _GCH == 0
    max_g = (n_chunks + _NW - 1) // _NW

    mesh = plsc.VectorSubcoreMesh(core_axis_name="c", subcore_axis_name="s")

    @functools.partial(
        pl.kernel,
        out_type=jax.ShapeDtypeStruct((e, d), jnp.float32),
        mesh=mesh,
        scratch_types=dict(
            idx_v=pltpu.VMEM((max_g, _GCH), jnp.int32),
            rows_v=[pltpu.VMEM((_GCH, d), jnp.float32) for _ in range(_GNBUF)],
            isem=pltpu.SemaphoreType.DMA,
            gsem=[pltpu.SemaphoreType.DMA for _ in range(_GNBUF)],
            wsem=[pltpu.SemaphoreType.DMA for _ in range(_GNBUF)],
        ),
    )
    def gather_k(xl_hbm, src_hbm, out_hbm, idx_v, rows_v, isem, gsem, wsem):
        wid = lax.axis_index("s") * 2 + lax.axis_index("c")

        def chunk_id(g):
            return g * _NW + wid

        def valid(g):
            return chunk_id(g) < n_chunks

        for g in range(max_g):
            @pl.when(valid(g))
            def _():
                pltpu.async_copy(
                    src_hbm.at[pl.ds(chunk_id(g) * _GCH, _GCH)],
                    idx_v.at[g], isem)
        for g in range(max_g):
            @pl.when(valid(g))
            def _():
                pltpu.make_async_copy(
                    src_hbm.at[pl.ds(chunk_id(g) * _GCH, _GCH)],
                    idx_v.at[g], isem).wait()

        def start_gather(g, slot):
            pltpu.async_copy(xl_hbm.at[idx_v.at[g]], rows_v[slot], gsem[slot])

        def wait_gather(g, slot):
            pltpu.make_async_copy(
                xl_hbm.at[idx_v.at[g]], rows_v[slot], gsem[slot]).wait()

        def out_slice(g):
            return out_hbm.at[pl.ds(chunk_id(g) * _GCH, _GCH)]

        def start_write(g, slot):
            pltpu.async_copy(rows_v[slot], out_slice(g), wsem[slot])

        def wait_write(g, slot):
            pltpu.make_async_copy(rows_v[slot], out_slice(g), wsem[slot]).wait()

        for g in range(max_g + 1):
            slot = g % _GNBUF
            if g < max_g:
                if g >= _GNBUF:
                    @pl.when(valid(g))
                    def _():
                        wait_write(g - _GNBUF, slot)
                @pl.when(valid(g))
                def _():
                    start_gather(g, slot)
            if g >= 1:
                pslot = (g - 1) % _GNBUF
                @pl.when(valid(g - 1))
                def _():
                    wait_gather(g - 1, pslot)
                    start_write(g - 1, pslot)
        for g in range(max(0, max_g - _GNBUF), max_g):
            @pl.when(valid(g))
            def _():
                wait_write(g, g % _GNBUF)

    return gather_k(xl, src)


def kernel(x, z, edge_src, edge_dst, edge_attr, edge_scalars,
           W_lin1, W_fc1, W_fc2, W_lin2_s, W_lin2_v, W_sc):
    n = x.shape[0]
    Wv_rep = jnp.repeat(W_lin2_v, 3, axis=1)
    R3 = jnp.tile(jnp.eye(3, dtype=jnp.float32), (1, 32))
    E32 = jnp.repeat(jnp.eye(32, dtype=jnp.float32), 3, axis=1)

    xl, sc = _node_stage(x, z, W_lin1, W_sc)
    xs = _sc_gather(xl, edge_src)
    msg_s, msg_v = _edge_stage(edge_scalars, xs, edge_attr,
                               W_fc1, W_fc2, W_lin2_s, Wv_rep, R3)
    agg_s = jax.ops.segment_sum(msg_s, edge_dst, num_segments=n)
    agg_v = jax.ops.segment_sum(msg_v, edge_dst, num_segments=n)
    return _final_stage(sc, agg_s, agg_v, z, E32)


# speedup vs baseline: 5.3186x; 2.8905x over previous
import functools
import math

import jax
import jax.numpy as jnp
from jax import lax
from jax.experimental import pallas as pl
from jax.experimental.pallas import tpu as pltpu
from jax.experimental.pallas import tpu_sc as plsc

_INV_SQRT128 = 1.0 / math.sqrt(128.0)
_INV_SQRT12 = 1.0 / math.sqrt(12.0)
_INV_SQRT100 = 1.0 / math.sqrt(100.0)
_INV_SQRT_NN = 1.0 / math.sqrt(16.0)

N_BLK = 2000
E_BLK = 2000


def _node_body(x_ref, z_ref, wl_ref, ws_ref, xl_ref, sc_ref):
    xz = x_ref[...] * z_ref[...]
    xl_ref[...] = (xz @ wl_ref[...]) * _INV_SQRT128
    sc_ref[...] = (xz @ ws_ref[...]) * _INV_SQRT128


def _node_stage(x, z, W_lin1, W_sc):
    n = x.shape[0]
    grid = (n // N_BLK,)
    return pl.pallas_call(
        _node_body,
        grid=grid,
        in_specs=[
            pl.BlockSpec((N_BLK, 128), lambda i: (i, 0)),
            pl.BlockSpec((N_BLK, 1), lambda i: (i, 0)),
            pl.BlockSpec((128, 128), lambda i: (0, 0)),
            pl.BlockSpec((128, 96), lambda i: (0, 0)),
        ],
        out_specs=[
            pl.BlockSpec((N_BLK, 128), lambda i: (i, 0)),
            pl.BlockSpec((N_BLK, 96), lambda i: (i, 0)),
        ],
        out_shape=[
            jax.ShapeDtypeStruct((n, 128), jnp.float32),
            jax.ShapeDtypeStruct((n, 96), jnp.float32),
        ],
    )(x, z, W_lin1, W_sc)


def _edge_body(es_ref, xs_ref, attr_ref, wf1_ref, wf2_ref, wls_ref, wvr_ref,
               r3_ref, msg_ref):
    h = jax.nn.silu((es_ref[...] @ wf1_ref[...]) * _INV_SQRT12)
    w = (h @ wf2_ref[...]) * _INV_SQRT100
    xs = xs_ref[...]
    sh0 = attr_ref[:, 0:1]
    sh1 = attr_ref[:, 1:4]
    t0 = xs * sh0 * w[:, :128]
    msg_ref[0] = (t0 @ wls_ref[...]) * _INV_SQRT128
    t1 = xs * w[:, 128:]
    msg_ref[1] = ((t1 @ wvr_ref[...]) * _INV_SQRT128) * (sh1 @ r3_ref[...])


def _edge_stage(es, xs, attr, W_fc1, W_fc2, W_lin2_s, Wv_rep, R3):
    e = es.shape[0]
    grid = (e // E_BLK,)
    return pl.pallas_call(
        _edge_body,
        grid=grid,
        in_specs=[
            pl.BlockSpec((E_BLK, 12), lambda i: (i, 0)),
            pl.BlockSpec((E_BLK, 128), lambda i: (i, 0)),
            pl.BlockSpec((E_BLK, 4), lambda i: (i, 0)),
            pl.BlockSpec((12, 100), lambda i: (0, 0)),
            pl.BlockSpec((100, 256), lambda i: (0, 0)),
            pl.BlockSpec((128, 128), lambda i: (0, 0)),
            pl.BlockSpec((128, 128), lambda i: (0, 0)),
            pl.BlockSpec((3, 128), lambda i: (0, 0)),
        ],
        out_specs=pl.BlockSpec((2, E_BLK, 128), lambda i: (0, i, 0)),
        out_shape=jax.ShapeDtypeStruct((2, e, 128), jnp.float32),
    )(es, xs, attr, W_fc1, W_fc2, W_lin2_s, Wv_rep, R3)


def _final_body(sc_ref, agg_ref, z_ref, e32_ref, out_ref):
    z = z_ref[...]
    s_out = sc_ref[...] + agg_ref[0][:, :96] * z * _INV_SQRT_NN
    scal = jax.nn.silu(s_out[:, :64])
    gates = jax.nn.sigmoid(s_out[:, 64:96])
    gated = (agg_ref[1][:, :96] * z * _INV_SQRT_NN) * (gates @ e32_ref[...])
    out_ref[:, :64] = scal
    out_ref[:, 64:160] = gated


def _final_stage(sc, agg, z, E32):
    n = sc.shape[0]
    grid = (n // N_BLK,)
    return pl.pallas_call(
        _final_body,
        grid=grid,
        in_specs=[
            pl.BlockSpec((N_BLK, 96), lambda i: (i, 0)),
            pl.BlockSpec((2, N_BLK, 128), lambda i: (0, i, 0)),
            pl.BlockSpec((N_BLK, 1), lambda i: (i, 0)),
            pl.BlockSpec((32, 96), lambda i: (0, 0)),
        ],
        out_specs=pl.BlockSpec((N_BLK, 160), lambda i: (i, 0)),
        out_shape=jax.ShapeDtypeStruct((n, 160), jnp.float32),
    )(sc, agg, z, E32)


_NW = 32
_GCH = 128
_GNBUF = 4


def _sc_gather(xl, src):
    e = src.shape[0]
    d = xl.shape[1]
    n_chunks = e // _GCH
    assert e % _GCH == 0
    max_g = (n_chunks + _NW - 1) // _NW

    mesh = plsc.VectorSubcoreMesh(core_axis_name="c", subcore_axis_name="s",
                                  num_cores=2, num_subcores=16)

    @functools.partial(
        pl.kernel,
        out_type=jax.ShapeDtypeStruct((e, d), jnp.float32),
        mesh=mesh,
        scratch_types=dict(
            idx_v=pltpu.VMEM((max_g, _GCH), jnp.int32),
            rows_v=[pltpu.VMEM((_GCH, d), jnp.float32) for _ in range(_GNBUF)],
            isem=pltpu.SemaphoreType.DMA,
            gsem=[pltpu.SemaphoreType.DMA for _ in range(_GNBUF)],
            wsem=[pltpu.SemaphoreType.DMA for _ in range(_GNBUF)],
        ),
    )
    def gather_k(xl_hbm, src_hbm, out_hbm, idx_v, rows_v, isem, gsem, wsem):
        wid = lax.axis_index("s") * 2 + lax.axis_index("c")

        def chunk_id(g):
            return g * _NW + wid

        def valid(g):
            return chunk_id(g) < n_chunks

        for g in range(max_g):
            @pl.when(valid(g))
            def _():
                pltpu.async_copy(
                    src_hbm.at[pl.ds(chunk_id(g) * _GCH, _GCH)],
                    idx_v.at[g], isem)
        for g in range(max_g):
            @pl.when(valid(g))
            def _():
                pltpu.make_async_copy(
                    src_hbm.at[pl.ds(chunk_id(g) * _GCH, _GCH)],
                    idx_v.at[g], isem).wait()

        def start_gather(g, slot):
            pltpu.async_copy(xl_hbm.at[idx_v.at[g]], rows_v[slot], gsem[slot])

        def wait_gather(g, slot):
            pltpu.make_async_copy(
                xl_hbm.at[idx_v.at[g]], rows_v[slot], gsem[slot]).wait()

        def out_slice(g):
            return out_hbm.at[pl.ds(chunk_id(g) * _GCH, _GCH)]

        def start_write(g, slot):
            pltpu.async_copy(rows_v[slot], out_slice(g), wsem[slot])

        def wait_write(g, slot):
            pltpu.make_async_copy(rows_v[slot], out_slice(g), wsem[slot]).wait()

        for g in range(max_g + 1):
            slot = g % _GNBUF
            if g < max_g:
                if g >= _GNBUF:
                    @pl.when(valid(g))
                    def _():
                        wait_write(g - _GNBUF, slot)
                @pl.when(valid(g))
                def _():
                    start_gather(g, slot)
            if g >= 1:
                pslot = (g - 1) % _GNBUF
                @pl.when(valid(g - 1))
                def _():
                    wait_gather(g - 1, pslot)
                    start_write(g - 1, pslot)
        for g in range(max(0, max_g - _GNBUF), max_g):
            @pl.when(valid(g))
            def _():
                wait_write(g, g % _GNBUF)

    return gather_k(xl, src)


_SCH = 128
_SNBUF = 2
_SLA = 1


def _sc_scatter(msg, dst, n):
    _, e, d = msg.shape
    n_chunks = e // _SCH
    assert e % _SCH == 0
    ns = 16
    max_g = (n_chunks + ns - 1) // ns
    nz = n // ns // 8 * 8
    nrem = n - nz * ns
    assert nrem % 8 == 0

    mesh = plsc.VectorSubcoreMesh(core_axis_name="c", subcore_axis_name="s",
                                  num_cores=2, num_subcores=16)

    @functools.partial(
        pl.kernel,
        out_type=jax.ShapeDtypeStruct((2, n, d), jnp.float32),
        mesh=mesh,
        scratch_types=dict(
            acc=pltpu.VMEM_SHARED((n, d), jnp.float32),
            idx_v=pltpu.VMEM((_SNBUF, _SCH), jnp.int32),
            msg_v=[pltpu.VMEM((_SCH, d), jnp.float32) for _ in range(_SNBUF)],
            isem=[pltpu.SemaphoreType.DMA for _ in range(_SNBUF)],
            msem=[pltpu.SemaphoreType.DMA for _ in range(_SNBUF)],
            zsem=pltpu.SemaphoreType.DMA,
        ),
    )
    def scatter_k(msg_hbm, dst_hbm, zero_hbm, out_hbm,
                  acc, idx_v, msg_v, isem, msem, zsem):
        cid = lax.axis_index("c")
        sid = lax.axis_index("s")

        pltpu.async_copy(zero_hbm.at[pl.ds(sid * nz, nz)],
                         acc.at[pl.ds(sid * nz, nz)], zsem)
        pltpu.make_async_copy(zero_hbm.at[pl.ds(sid * nz, nz)],
                              acc.at[pl.ds(sid * nz, nz)], zsem).wait()
        if nrem:
            @pl.when(sid == ns - 1)
            def _():
                pltpu.async_copy(zero_hbm.at[pl.ds(ns * nz, nrem)],
                                 acc.at[pl.ds(ns * nz, nrem)], zsem)
                pltpu.make_async_copy(zero_hbm.at[pl.ds(ns * nz, nrem)],
                                      acc.at[pl.ds(ns * nz, nrem)], zsem).wait()
        plsc.subcore_barrier()

        def chunk_id(g):
            return g * ns + sid

        def valid(g):
            return chunk_id(g) < n_chunks

        def idx_slice(g):
            return dst_hbm.at[pl.ds(chunk_id(g) * _SCH, _SCH)]

        def msg_slice(g):
            return msg_hbm.at[cid, pl.ds(chunk_id(g) * _SCH, _SCH)]

        def start_loads(g, slot):
            pltpu.async_copy(idx_slice(g), idx_v.at[slot], isem[slot])
            pltpu.async_copy(msg_slice(g), msg_v[slot], msem[slot])

        def wait_loads(g, slot):
            pltpu.make_async_copy(idx_slice(g), idx_v.at[slot], isem[slot]).wait()
            pltpu.make_async_copy(msg_slice(g), msg_v[slot], msem[slot]).wait()

        @pl.when(valid(0))
        def _():
            start_loads(0, 0)
        for g in range(max_g):
            slot = g % _SNBUF
            @pl.when(valid(g))
            def _():
                wait_loads(g, slot)
            nx = g + 1
            if nx < max_g:
                @pl.when(valid(nx))
                def _():
                    start_loads(nx, nx % _SNBUF)
            @pl.when(valid(g))
            def _():
                pltpu.sync_copy(msg_v[slot], acc.at[idx_v.at[slot]], add=True)

        plsc.subcore_barrier()
        pltpu.async_copy(acc.at[pl.ds(sid * nz, nz)],
                         out_hbm.at[cid, pl.ds(sid * nz, nz)], zsem)
        pltpu.make_async_copy(acc.at[pl.ds(sid * nz, nz)],
                              out_hbm.at[cid, pl.ds(sid * nz, nz)], zsem).wait()
        if nrem:
            @pl.when(sid == ns - 1)
            def _():
                pltpu.async_copy(acc.at[pl.ds(ns * nz, nrem)],
                                 out_hbm.at[cid, pl.ds(ns * nz, nrem)], zsem)
                pltpu.make_async_copy(
                    acc.at[pl.ds(ns * nz, nrem)],
                    out_hbm.at[cid, pl.ds(ns * nz, nrem)], zsem).wait()

    zero = jnp.zeros((n, d), jnp.float32)
    return scatter_k(msg, dst, zero)


def kernel(x, z, edge_src, edge_dst, edge_attr, edge_scalars,
           W_lin1, W_fc1, W_fc2, W_lin2_s, W_lin2_v, W_sc):
    n = x.shape[0]
    Wv_rep = jnp.pad(jnp.repeat(W_lin2_v, 3, axis=1), ((0, 0), (0, 32)))
    Wls_pad = jnp.pad(W_lin2_s, ((0, 0), (0, 32)))
    R3 = jnp.pad(jnp.tile(jnp.eye(3, dtype=jnp.float32), (1, 32)),
                 ((0, 0), (0, 32)))
    E32 = jnp.repeat(jnp.eye(32, dtype=jnp.float32), 3, axis=1)

    e_n = edge_src.shape[0]
    xl, sc = _node_stage(x, z, W_lin1, W_sc)
    xs = _sc_gather(xl, edge_src)
    msg = _edge_stage(edge_scalars, xs, edge_attr,
                      W_fc1, W_fc2, Wls_pad, Wv_rep, R3)
    agg = _sc_scatter(msg, edge_dst, n)
    return _final_stage(sc, agg, z, E32)


# speedup vs baseline: 5.3660x; 1.0089x over previous
import functools
import math

import jax
import jax.numpy as jnp
from jax import lax
from jax.experimental import pallas as pl
from jax.experimental.pallas import tpu as pltpu
from jax.experimental.pallas import tpu_sc as plsc

_INV_SQRT128 = 1.0 / math.sqrt(128.0)
_INV_SQRT12 = 1.0 / math.sqrt(12.0)
_INV_SQRT100 = 1.0 / math.sqrt(100.0)
_INV_SQRT_NN = 1.0 / math.sqrt(16.0)

N_BLK = 2000
E_BLK = 2000


def _node_body(x_ref, z_ref, wl_ref, ws_ref, xl_ref, sc_ref):
    xz = x_ref[...] * z_ref[...]
    xl_ref[...] = (xz @ wl_ref[...]) * _INV_SQRT128
    sc_ref[...] = (xz @ ws_ref[...]) * _INV_SQRT128


def _node_stage(x, z, W_lin1, W_sc):
    n = x.shape[0]
    grid = (n // N_BLK,)
    return pl.pallas_call(
        _node_body,
        grid=grid,
        in_specs=[
            pl.BlockSpec((N_BLK, 128), lambda i: (i, 0)),
            pl.BlockSpec((N_BLK, 1), lambda i: (i, 0)),
            pl.BlockSpec((128, 128), lambda i: (0, 0)),
            pl.BlockSpec((128, 96), lambda i: (0, 0)),
        ],
        out_specs=[
            pl.BlockSpec((N_BLK, 128), lambda i: (i, 0)),
            pl.BlockSpec((N_BLK, 96), lambda i: (i, 0)),
        ],
        out_shape=[
            jax.ShapeDtypeStruct((n, 128), jnp.float32),
            jax.ShapeDtypeStruct((n, 96), jnp.float32),
        ],
    )(x, z, W_lin1, W_sc)


def _edge_body(es_ref, xs_ref, attr_ref, wf1_ref, wf2_ref, wvr_ref,
               r3_ref, msg_ref):
    f32 = jnp.float32
    h = jax.nn.silu(
        jnp.dot(es_ref[...].astype(jnp.bfloat16), wf1_ref[...],
                preferred_element_type=f32) * _INV_SQRT12)
    w = jnp.dot(h.astype(jnp.bfloat16), wf2_ref[...],
                preferred_element_type=f32) * _INV_SQRT100
    xs = xs_ref[...]
    sh0 = attr_ref[:, 0:1]
    sh1 = attr_ref[:, 1:4]
    msg_ref[0] = xs * sh0 * w[:, :128]
    t1 = xs * w[:, 128:]
    msg_ref[1] = (jnp.dot(t1.astype(jnp.bfloat16), wvr_ref[...],
                          preferred_element_type=f32) * _INV_SQRT128
                  ) * (sh1 @ r3_ref[...])


def _edge_stage(es, xs, attr, W_fc1, W_fc2, Wv_rep, R3):
    e = es.shape[0]
    grid = (e // E_BLK,)
    return pl.pallas_call(
        _edge_body,
        grid=grid,
        in_specs=[
            pl.BlockSpec((E_BLK, 12), lambda i: (i, 0)),
            pl.BlockSpec((E_BLK, 128), lambda i: (i, 0)),
            pl.BlockSpec((E_BLK, 4), lambda i: (i, 0)),
            pl.BlockSpec((12, 100), lambda i: (0, 0)),
            pl.BlockSpec((100, 256), lambda i: (0, 0)),
            pl.BlockSpec((128, 128), lambda i: (0, 0)),
            pl.BlockSpec((3, 128), lambda i: (0, 0)),
        ],
        out_specs=pl.BlockSpec((2, E_BLK, 128), lambda i: (0, i, 0)),
        out_shape=jax.ShapeDtypeStruct((2, e, 128), jnp.float32),
    )(es, xs, attr, W_fc1, W_fc2, Wv_rep, R3)


def _final_body(sc_ref, agg_ref, z_ref, wls_ref, e32_ref, out_ref):
    z = z_ref[...]
    s_out = sc_ref[...] + (agg_ref[0] @ wls_ref[...]) * (
        z * (_INV_SQRT128 * _INV_SQRT_NN))
    scal = jax.nn.silu(s_out[:, :64])
    gates = jax.nn.sigmoid(s_out[:, 64:96])
    gated = (agg_ref[1][:, :96] * z * _INV_SQRT_NN) * (gates @ e32_ref[...])
    out_ref[:, :64] = scal
    out_ref[:, 64:160] = gated


def _final_stage(sc, agg, z, W_lin2_s, E32):
    n = sc.shape[0]
    grid = (n // N_BLK,)
    return pl.pallas_call(
        _final_body,
        grid=grid,
        in_specs=[
            pl.BlockSpec((N_BLK, 96), lambda i: (i, 0)),
            pl.BlockSpec((2, N_BLK, 128), lambda i: (0, i, 0)),
            pl.BlockSpec((N_BLK, 1), lambda i: (i, 0)),
            pl.BlockSpec((128, 96), lambda i: (0, 0)),
            pl.BlockSpec((32, 96), lambda i: (0, 0)),
        ],
        out_specs=pl.BlockSpec((N_BLK, 160), lambda i: (i, 0)),
        out_shape=jax.ShapeDtypeStruct((n, 160), jnp.float32),
    )(sc, agg, z, W_lin2_s, E32)


_NW = 32
_GCH = 128
_GNBUF = 4


def _sc_gather(xl, src):
    e = src.shape[0]
    d = xl.shape[1]
    n_chunks = e // _GCH
    assert e % _GCH == 0
    max_g = (n_chunks + _NW - 1) // _NW

    mesh = plsc.VectorSubcoreMesh(core_axis_name="c", subcore_axis_name="s",
                                  num_cores=2, num_subcores=16)

    @functools.partial(
        pl.kernel,
        out_type=jax.ShapeDtypeStruct((e, d), jnp.float32),
        mesh=mesh,
        scratch_types=dict(
            idx_v=pltpu.VMEM((max_g, _GCH), jnp.int32),
            rows_v=[pltpu.VMEM((_GCH, d), jnp.float32) for _ in range(_GNBUF)],
            isem=pltpu.SemaphoreType.DMA,
            gsem=[pltpu.SemaphoreType.DMA for _ in range(_GNBUF)],
            wsem=[pltpu.SemaphoreType.DMA for _ in range(_GNBUF)],
        ),
    )
    def gather_k(xl_hbm, src_hbm, out_hbm, idx_v, rows_v, isem, gsem, wsem):
        wid = lax.axis_index("s") * 2 + lax.axis_index("c")

        def chunk_id(g):
            return g * _NW + wid

        def valid(g):
            return chunk_id(g) < n_chunks

        for g in range(max_g):
            @pl.when(valid(g))
            def _():
                pltpu.async_copy(
                    src_hbm.at[pl.ds(chunk_id(g) * _GCH, _GCH)],
                    idx_v.at[g], isem)
        for g in range(max_g):
            @pl.when(valid(g))
            def _():
                pltpu.make_async_copy(
                    src_hbm.at[pl.ds(chunk_id(g) * _GCH, _GCH)],
                    idx_v.at[g], isem).wait()

        def start_gather(g, slot):
            pltpu.async_copy(xl_hbm.at[idx_v.at[g]], rows_v[slot], gsem[slot])

        def wait_gather(g, slot):
            pltpu.make_async_copy(
                xl_hbm.at[idx_v.at[g]], rows_v[slot], gsem[slot]).wait()

        def out_slice(g):
            return out_hbm.at[pl.ds(chunk_id(g) * _GCH, _GCH)]

        def start_write(g, slot):
            pltpu.async_copy(rows_v[slot], out_slice(g), wsem[slot])

        def wait_write(g, slot):
            pltpu.make_async_copy(rows_v[slot], out_slice(g), wsem[slot]).wait()

        for g in range(max_g + 1):
            slot = g % _GNBUF
            if g < max_g:
                if g >= _GNBUF:
                    @pl.when(valid(g))
                    def _():
                        wait_write(g - _GNBUF, slot)
                @pl.when(valid(g))
                def _():
                    start_gather(g, slot)
            if g >= 1:
                pslot = (g - 1) % _GNBUF
                @pl.when(valid(g - 1))
                def _():
                    wait_gather(g - 1, pslot)
                    start_write(g - 1, pslot)
        for g in range(max(0, max_g - _GNBUF), max_g):
            @pl.when(valid(g))
            def _():
                wait_write(g, g % _GNBUF)

    return gather_k(xl, src)


_SCH = 128
_SNBUF = 2
_SLA = 1


def _sc_scatter(msg, dst, n):
    _, e, d = msg.shape
    n_chunks = e // _SCH
    assert e % _SCH == 0
    ns = 16
    max_g = (n_chunks + ns - 1) // ns
    nz = n // ns // 8 * 8
    nrem = n - nz * ns
    assert nrem % 8 == 0

    mesh = plsc.VectorSubcoreMesh(core_axis_name="c", subcore_axis_name="s",
                                  num_cores=2, num_subcores=16)

    @functools.partial(
        pl.kernel,
        out_type=jax.ShapeDtypeStruct((2, n, d), jnp.float32),
        mesh=mesh,
        scratch_types=dict(
            acc=pltpu.VMEM_SHARED((n, d), jnp.float32),
            idx_v=pltpu.VMEM((_SNBUF, _SCH), jnp.int32),
            msg_v=[pltpu.VMEM((_SCH, d), jnp.float32) for _ in range(_SNBUF)],
            isem=[pltpu.SemaphoreType.DMA for _ in range(_SNBUF)],
            msem=[pltpu.SemaphoreType.DMA for _ in range(_SNBUF)],
            zsem=pltpu.SemaphoreType.DMA,
        ),
    )
    def scatter_k(msg_hbm, dst_hbm, zero_hbm, out_hbm,
                  acc, idx_v, msg_v, isem, msem, zsem):
        cid = lax.axis_index("c")
        sid = lax.axis_index("s")

        pltpu.async_copy(zero_hbm.at[pl.ds(sid * nz, nz)],
                         acc.at[pl.ds(sid * nz, nz)], zsem)
        pltpu.make_async_copy(zero_hbm.at[pl.ds(sid * nz, nz)],
                              acc.at[pl.ds(sid * nz, nz)], zsem).wait()
        if nrem:
            @pl.when(sid == ns - 1)
            def _():
                pltpu.async_copy(zero_hbm.at[pl.ds(ns * nz, nrem)],
                                 acc.at[pl.ds(ns * nz, nrem)], zsem)
                pltpu.make_async_copy(zero_hbm.at[pl.ds(ns * nz, nrem)],
                                      acc.at[pl.ds(ns * nz, nrem)], zsem).wait()
        plsc.subcore_barrier()

        def chunk_id(g):
            return g * ns + sid

        def valid(g):
            return chunk_id(g) < n_chunks

        def idx_slice(g):
            return dst_hbm.at[pl.ds(chunk_id(g) * _SCH, _SCH)]

        def msg_slice(g):
            return msg_hbm.at[cid, pl.ds(chunk_id(g) * _SCH, _SCH)]

        def start_loads(g, slot):
            pltpu.async_copy(idx_slice(g), idx_v.at[slot], isem[slot])
            pltpu.async_copy(msg_slice(g), msg_v[slot], msem[slot])

        def wait_loads(g, slot):
            pltpu.make_async_copy(idx_slice(g), idx_v.at[slot], isem[slot]).wait()
            pltpu.make_async_copy(msg_slice(g), msg_v[slot], msem[slot]).wait()

        @pl.when(valid(0))
        def _():
            start_loads(0, 0)
        for g in range(max_g):
            slot = g % _SNBUF
            @pl.when(valid(g))
            def _():
                wait_loads(g, slot)
            nx = g + 1
            if nx < max_g:
                @pl.when(valid(nx))
                def _():
                    start_loads(nx, nx % _SNBUF)
            @pl.when(valid(g))
            def _():
                pltpu.sync_copy(msg_v[slot], acc.at[idx_v.at[slot]], add=True)

        plsc.subcore_barrier()
        pltpu.async_copy(acc.at[pl.ds(sid * nz, nz)],
                         out_hbm.at[cid, pl.ds(sid * nz, nz)], zsem)
        pltpu.make_async_copy(acc.at[pl.ds(sid * nz, nz)],
                              out_hbm.at[cid, pl.ds(sid * nz, nz)], zsem).wait()
        if nrem:
            @pl.when(sid == ns - 1)
            def _():
                pltpu.async_copy(acc.at[pl.ds(ns * nz, nrem)],
                                 out_hbm.at[cid, pl.ds(ns * nz, nrem)], zsem)
                pltpu.make_async_copy(
                    acc.at[pl.ds(ns * nz, nrem)],
                    out_hbm.at[cid, pl.ds(ns * nz, nrem)], zsem).wait()

    zero = jnp.zeros((n, d), jnp.float32)
    return scatter_k(msg, dst, zero)


def kernel(x, z, edge_src, edge_dst, edge_attr, edge_scalars,
           W_lin1, W_fc1, W_fc2, W_lin2_s, W_lin2_v, W_sc):
    n = x.shape[0]
    bf16 = jnp.bfloat16
    Wv_rep = jnp.pad(jnp.repeat(W_lin2_v, 3, axis=1),
                     ((0, 0), (0, 32))).astype(bf16)
    R3 = jnp.pad(jnp.tile(jnp.eye(3, dtype=jnp.float32), (1, 32)),
                 ((0, 0), (0, 32)))
    E32 = jnp.repeat(jnp.eye(32, dtype=jnp.float32), 3, axis=1)

    xl, sc = _node_stage(x, z, W_lin1, W_sc)
    xs = _sc_gather(xl, edge_src)
    msg = _edge_stage(edge_scalars, xs, edge_attr,
                      W_fc1.astype(bf16), W_fc2.astype(bf16), Wv_rep, R3)
    agg = _sc_scatter(msg, edge_dst, n)
    return _final_stage(sc, agg, z, W_lin2_s, E32)


# speedup vs baseline: 5.6355x; 1.0502x over previous
import functools
import math

import jax
import jax.numpy as jnp
from jax import lax
from jax.experimental import pallas as pl
from jax.experimental.pallas import tpu as pltpu
from jax.experimental.pallas import tpu_sc as plsc

_INV_SQRT128 = 1.0 / math.sqrt(128.0)
_INV_SQRT12 = 1.0 / math.sqrt(12.0)
_INV_SQRT100 = 1.0 / math.sqrt(100.0)
_INV_SQRT_NN = 1.0 / math.sqrt(16.0)

N_BLK = 2000
E_BLK = 5000


def _node_body(x_ref, z_ref, wl_ref, ws_ref, xl_ref, sc_ref):
    xz = x_ref[...] * z_ref[...]
    xl_ref[...] = (xz @ wl_ref[...]) * _INV_SQRT128
    sc_ref[...] = (xz @ ws_ref[...]) * _INV_SQRT128


def _node_stage(x, z, W_lin1, W_sc):
    n = x.shape[0]
    grid = (n // N_BLK,)
    return pl.pallas_call(
        _node_body,
        grid=grid,
        in_specs=[
            pl.BlockSpec((N_BLK, 128), lambda i: (i, 0)),
            pl.BlockSpec((N_BLK, 1), lambda i: (i, 0)),
            pl.BlockSpec((128, 128), lambda i: (0, 0)),
            pl.BlockSpec((128, 96), lambda i: (0, 0)),
        ],
        out_specs=[
            pl.BlockSpec((N_BLK, 128), lambda i: (i, 0)),
            pl.BlockSpec((N_BLK, 96), lambda i: (i, 0)),
        ],
        out_shape=[
            jax.ShapeDtypeStruct((n, 128), jnp.float32),
            jax.ShapeDtypeStruct((n, 96), jnp.float32),
        ],
    )(x, z, W_lin1, W_sc)


def _edge_body(es_ref, xs_ref, attr_ref, wf1_ref, wf2_ref, wvr_ref,
               r3_ref, msg_ref):
    f32 = jnp.float32
    h = jax.nn.silu(
        jnp.dot(es_ref[...].astype(jnp.bfloat16), wf1_ref[...],
                preferred_element_type=f32) * _INV_SQRT12)
    w = jnp.dot(h.astype(jnp.bfloat16), wf2_ref[...],
                preferred_element_type=f32) * _INV_SQRT100
    xs = xs_ref[...]
    sh0 = attr_ref[:, 0:1]
    sh1 = attr_ref[:, 1:4]
    msg_ref[0] = xs * sh0 * w[:, :128]
    t1 = xs * w[:, 128:]
    msg_ref[1] = (jnp.dot(t1.astype(jnp.bfloat16), wvr_ref[...],
                          preferred_element_type=f32) * _INV_SQRT128
                  ) * (sh1 @ r3_ref[...])


def _edge_stage(es, xs, attr, W_fc1, W_fc2, Wv_rep, R3):
    e = es.shape[0]
    grid = (e // E_BLK,)
    return pl.pallas_call(
        _edge_body,
        grid=grid,
        in_specs=[
            pl.BlockSpec((E_BLK, 12), lambda i: (i, 0)),
            pl.BlockSpec((E_BLK, 128), lambda i: (i, 0)),
            pl.BlockSpec((E_BLK, 4), lambda i: (i, 0)),
            pl.BlockSpec((12, 100), lambda i: (0, 0)),
            pl.BlockSpec((100, 256), lambda i: (0, 0)),
            pl.BlockSpec((128, 128), lambda i: (0, 0)),
            pl.BlockSpec((3, 128), lambda i: (0, 0)),
        ],
        out_specs=pl.BlockSpec((2, E_BLK, 128), lambda i: (0, i, 0)),
        out_shape=jax.ShapeDtypeStruct((2, e, 128), jnp.float32),
    )(es, xs, attr, W_fc1, W_fc2, Wv_rep, R3)


def _final_body(sc_ref, agg_ref, z_ref, wls_ref, e32_ref, out_ref):
    z = z_ref[...]
    s_out = sc_ref[...] + (agg_ref[0] @ wls_ref[...]) * (
        z * (_INV_SQRT128 * _INV_SQRT_NN))
    scal = jax.nn.silu(s_out[:, :64])
    gates = jax.nn.sigmoid(s_out[:, 64:96])
    gated = (agg_ref[1][:, :96] * z * _INV_SQRT_NN) * (gates @ e32_ref[...])
    out_ref[:, :64] = scal
    out_ref[:, 64:160] = gated


def _final_stage(sc, agg, z, W_lin2_s, E32):
    n = sc.shape[0]
    grid = (n // N_BLK,)
    return pl.pallas_call(
        _final_body,
        grid=grid,
        in_specs=[
            pl.BlockSpec((N_BLK, 96), lambda i: (i, 0)),
            pl.BlockSpec((2, N_BLK, 128), lambda i: (0, i, 0)),
            pl.BlockSpec((N_BLK, 1), lambda i: (i, 0)),
            pl.BlockSpec((128, 96), lambda i: (0, 0)),
            pl.BlockSpec((32, 96), lambda i: (0, 0)),
        ],
        out_specs=pl.BlockSpec((N_BLK, 160), lambda i: (i, 0)),
        out_shape=jax.ShapeDtypeStruct((n, 160), jnp.float32),
    )(sc, agg, z, W_lin2_s, E32)


_NW = 32
_GCH = 128
_GNBUF = 6


def _sc_gather(xl, src):
    e = src.shape[0]
    d = xl.shape[1]
    n_chunks = e // _GCH
    assert e % _GCH == 0
    max_g = (n_chunks + _NW - 1) // _NW

    mesh = plsc.VectorSubcoreMesh(core_axis_name="c", subcore_axis_name="s",
                                  num_cores=2, num_subcores=16)

    @functools.partial(
        pl.kernel,
        out_type=jax.ShapeDtypeStruct((e, d), jnp.float32),
        mesh=mesh,
        scratch_types=dict(
            idx_v=pltpu.VMEM((max_g, _GCH), jnp.int32),
            rows_v=[pltpu.VMEM((_GCH, d), jnp.float32) for _ in range(_GNBUF)],
            isem=pltpu.SemaphoreType.DMA,
            gsem=[pltpu.SemaphoreType.DMA for _ in range(_GNBUF)],
            wsem=[pltpu.SemaphoreType.DMA for _ in range(_GNBUF)],
        ),
    )
    def gather_k(xl_hbm, src_hbm, out_hbm, idx_v, rows_v, isem, gsem, wsem):
        wid = lax.axis_index("s") * 2 + lax.axis_index("c")

        def chunk_id(g):
            return g * _NW + wid

        def valid(g):
            return chunk_id(g) < n_chunks

        for g in range(max_g):
            @pl.when(valid(g))
            def _():
                pltpu.async_copy(
                    src_hbm.at[pl.ds(chunk_id(g) * _GCH, _GCH)],
                    idx_v.at[g], isem)
        for g in range(max_g):
            @pl.when(valid(g))
            def _():
                pltpu.make_async_copy(
                    src_hbm.at[pl.ds(chunk_id(g) * _GCH, _GCH)],
                    idx_v.at[g], isem).wait()

        def start_gather(g, slot):
            pltpu.async_copy(xl_hbm.at[idx_v.at[g]], rows_v[slot], gsem[slot])

        def wait_gather(g, slot):
            pltpu.make_async_copy(
                xl_hbm.at[idx_v.at[g]], rows_v[slot], gsem[slot]).wait()

        def out_slice(g):
            return out_hbm.at[pl.ds(chunk_id(g) * _GCH, _GCH)]

        def start_write(g, slot):
            pltpu.async_copy(rows_v[slot], out_slice(g), wsem[slot])

        def wait_write(g, slot):
            pltpu.make_async_copy(rows_v[slot], out_slice(g), wsem[slot]).wait()

        for g in range(max_g + 1):
            slot = g % _GNBUF
            if g < max_g:
                if g >= _GNBUF:
                    @pl.when(valid(g))
                    def _():
                        wait_write(g - _GNBUF, slot)
                @pl.when(valid(g))
                def _():
                    start_gather(g, slot)
            if g >= 1:
                pslot = (g - 1) % _GNBUF
                @pl.when(valid(g - 1))
                def _():
                    wait_gather(g - 1, pslot)
                    start_write(g - 1, pslot)
        for g in range(max(0, max_g - _GNBUF), max_g):
            @pl.when(valid(g))
            def _():
                wait_write(g, g % _GNBUF)

    return gather_k(xl, src)


_SCH = 128
_SNBUF = 2
_SLA = 1


def _sc_scatter(msg, dst, n):
    _, e, d = msg.shape
    n_chunks = e // _SCH
    assert e % _SCH == 0
    ns = 16
    max_g = (n_chunks + ns - 1) // ns
    nz = n // ns // 8 * 8
    nrem = n - nz * ns
    assert nrem % 8 == 0

    mesh = plsc.VectorSubcoreMesh(core_axis_name="c", subcore_axis_name="s",
                                  num_cores=2, num_subcores=16)

    @functools.partial(
        pl.kernel,
        out_type=jax.ShapeDtypeStruct((2, n, d), jnp.float32),
        mesh=mesh,
        scratch_types=dict(
            acc=pltpu.VMEM_SHARED((n, d), jnp.float32),
            idx_v=pltpu.VMEM((_SNBUF, _SCH), jnp.int32),
            msg_v=[pltpu.VMEM((_SCH, d), jnp.float32) for _ in range(_SNBUF)],
            isem=[pltpu.SemaphoreType.DMA for _ in range(_SNBUF)],
            msem=[pltpu.SemaphoreType.DMA for _ in range(_SNBUF)],
            zsem=pltpu.SemaphoreType.DMA,
        ),
    )
    def scatter_k(msg_hbm, dst_hbm, zero_hbm, out_hbm,
                  acc, idx_v, msg_v, isem, msem, zsem):
        cid = lax.axis_index("c")
        sid = lax.axis_index("s")

        pltpu.async_copy(zero_hbm.at[pl.ds(sid * nz, nz)],
                         acc.at[pl.ds(sid * nz, nz)], zsem)
        pltpu.make_async_copy(zero_hbm.at[pl.ds(sid * nz, nz)],
                              acc.at[pl.ds(sid * nz, nz)], zsem).wait()
        if nrem:
            @pl.when(sid == ns - 1)
            def _():
                pltpu.async_copy(zero_hbm.at[pl.ds(ns * nz, nrem)],
                                 acc.at[pl.ds(ns * nz, nrem)], zsem)
                pltpu.make_async_copy(zero_hbm.at[pl.ds(ns * nz, nrem)],
                                      acc.at[pl.ds(ns * nz, nrem)], zsem).wait()
        plsc.subcore_barrier()

        def chunk_id(g):
            return g * ns + sid

        def valid(g):
            return chunk_id(g) < n_chunks

        def idx_slice(g):
            return dst_hbm.at[pl.ds(chunk_id(g) * _SCH, _SCH)]

        def msg_slice(g):
            return msg_hbm.at[cid, pl.ds(chunk_id(g) * _SCH, _SCH)]

        def start_loads(g, slot):
            pltpu.async_copy(idx_slice(g), idx_v.at[slot], isem[slot])
            pltpu.async_copy(msg_slice(g), msg_v[slot], msem[slot])

        def wait_loads(g, slot):
            pltpu.make_async_copy(idx_slice(g), idx_v.at[slot], isem[slot]).wait()
            pltpu.make_async_copy(msg_slice(g), msg_v[slot], msem[slot]).wait()

        @pl.when(valid(0))
        def _():
            start_loads(0, 0)
        for g in range(max_g):
            slot = g % _SNBUF
            @pl.when(valid(g))
            def _():
                wait_loads(g, slot)
            nx = g + 1
            if nx < max_g:
                @pl.when(valid(nx))
                def _():
                    start_loads(nx, nx % _SNBUF)
            @pl.when(valid(g))
            def _():
                pltpu.sync_copy(msg_v[slot], acc.at[idx_v.at[slot]], add=True)

        plsc.subcore_barrier()
        pltpu.async_copy(acc.at[pl.ds(sid * nz, nz)],
                         out_hbm.at[cid, pl.ds(sid * nz, nz)], zsem)
        pltpu.make_async_copy(acc.at[pl.ds(sid * nz, nz)],
                              out_hbm.at[cid, pl.ds(sid * nz, nz)], zsem).wait()
        if nrem:
            @pl.when(sid == ns - 1)
            def _():
                pltpu.async_copy(acc.at[pl.ds(ns * nz, nrem)],
                                 out_hbm.at[cid, pl.ds(ns * nz, nrem)], zsem)
                pltpu.make_async_copy(
                    acc.at[pl.ds(ns * nz, nrem)],
                    out_hbm.at[cid, pl.ds(ns * nz, nrem)], zsem).wait()

    zero = jnp.zeros((n, d), jnp.float32)
    return scatter_k(msg, dst, zero)


def kernel(x, z, edge_src, edge_dst, edge_attr, edge_scalars,
           W_lin1, W_fc1, W_fc2, W_lin2_s, W_lin2_v, W_sc):
    n = x.shape[0]
    bf16 = jnp.bfloat16
    Wv_rep = jnp.pad(jnp.repeat(W_lin2_v, 3, axis=1),
                     ((0, 0), (0, 32))).astype(bf16)
    R3 = jnp.pad(jnp.tile(jnp.eye(3, dtype=jnp.float32), (1, 32)),
                 ((0, 0), (0, 32)))
    E32 = jnp.repeat(jnp.eye(32, dtype=jnp.float32), 3, axis=1)

    xl, sc = _node_stage(x, z, W_lin1, W_sc)
    xs = _sc_gather(xl, edge_src)
    msg = _edge_stage(edge_scalars, xs, edge_attr,
                      W_fc1.astype(bf16), W_fc2.astype(bf16), Wv_rep, R3)
    agg = _sc_scatter(msg, edge_dst, n)
    return _final_stage(sc, agg, z, W_lin2_s, E32)
